# baseline (speedup 1.0000x reference)
import functools

import jax
import jax.numpy as jnp
from jax import lax
from jax.experimental import pallas as pl
from jax.experimental.pallas import tpu as pltpu
from jax.experimental.pallas import tpu_sc as plsc

N = 10000
E = 320000
D = 128
C = 16

NC = 2
NS = 16
NW = NC * NS
K = 128
CH = 80
QC = 16
EPAD = NW * CH * K
TOTALC = NW * CH

FAST_CID = 0
CHF = 160
CHS = 0
PF = CHF // QC
PS = CHS // QC
NP = 10112
RPS = NP // NS

RC = 1000
NCH = N // RC

_HI = jax.lax.Precision.HIGHEST

_mesh = plsc.VectorSubcoreMesh(
    core_axis_name="c", subcore_axis_name="s", num_cores=NC, num_subcores=NS
)


@functools.partial(
    pl.kernel,
    out_type=jax.ShapeDtypeStruct((NC * NP, D), jnp.float32),
    mesh=_mesh,
    scratch_types=[
        pltpu.VMEM((CH, K), jnp.int32),
        pltpu.VMEM((K, D), jnp.float32),
        pltpu.VMEM_SHARED((NP, D), jnp.float32),
        pltpu.SemaphoreType.DMA,
    ],
)
def _sc_count(dst_hbm, ones_hbm, z16_hbm, out_hbm, dst_v, ones_v, acc, sem):
    cid = lax.axis_index("c")
    sid = lax.axis_index("s")
    wid = sid * NC + cid
    pltpu.async_copy(dst_hbm.at[wid], dst_v, sem).wait()
    pltpu.async_copy(ones_hbm, ones_v, sem).wait()
    row0 = sid * RPS
    pltpu.async_copy(
        z16_hbm.at[pl.ds(row0, RPS)], acc.at[pl.ds(row0, RPS)], sem
    ).wait()
    plsc.subcore_barrier()

    @pl.loop(0, CH)
    def _(j):
        pltpu.sync_copy(ones_v, acc.at[dst_v.at[j]], add=True)

    plsc.subcore_barrier()
    pltpu.sync_copy(
        acc.at[pl.ds(row0, RPS)], out_hbm.at[pl.ds(cid * NP + row0, RPS)]
    )


NB = 2
GRP = QC // NB


@functools.partial(
    pl.kernel,
    out_type=jax.ShapeDtypeStruct((NC * NP, D), jnp.float32),
    mesh=_mesh,
    scratch_types=[
        pltpu.VMEM((QC, K), jnp.int32),
        pltpu.VMEM((QC, K), jnp.int32),
        pltpu.VMEM((NB, K, D), jnp.float32),
        pltpu.VMEM_SHARED((NP, D), jnp.float32),
        pltpu.SemaphoreType.DMA,
        pltpu.SemaphoreType.DMA,
        pltpu.SemaphoreType.DMA,
        pltpu.SemaphoreType.DMA,
        pltpu.SemaphoreType.DMA,
        pltpu.SemaphoreType.DMA,
        pltpu.SemaphoreType.DMA,
        pltpu.SemaphoreType.DMA,
        pltpu.SemaphoreType.DMA,
    ],
)
def _sc_agg(h_hbm, src_hbm, dst_hbm, zd_hbm, out_hbm,
            src_v, dst_v, buf, acc,
            sg0, sg1, sg2, sg3, ss0, ss1, ss2, ss3, sem):
    cid = lax.axis_index("c")
    sid = lax.axis_index("s")
    sg = (sg0, sg1, sg2, sg3)
    ss = (ss0, ss1, ss2, ss3)
    is_fast = cid == FAST_CID
    nph = jnp.where(is_fast, PF, PS)
    base = jnp.where(is_fast, sid * CHF, NS * CHF + sid * CHS)
    row0 = sid * RPS
    pltpu.async_copy(
        zd_hbm.at[pl.ds(row0, RPS)], acc.at[pl.ds(row0, RPS)], sem
    ).wait()
    plsc.subcore_barrier()

    @pl.loop(0, nph)
    def _(p):
        c0 = base + p * QC
        pltpu.async_copy(src_hbm.at[pl.ds(c0, QC)], src_v, sem)
        pltpu.async_copy(dst_hbm.at[pl.ds(c0, QC)], dst_v, sem)
        pltpu.make_async_copy(src_hbm.at[pl.ds(c0, QC)], src_v, sem).wait()
        pltpu.make_async_copy(dst_hbm.at[pl.ds(c0, QC)], dst_v, sem).wait()
        for k in range(NB):
            pltpu.async_copy(h_hbm.at[src_v.at[k]], buf.at[k], sg[k])

        @pl.loop(0, GRP - 1)
        def _(i):
            j0 = i * NB
            for k in range(NB):
                pltpu.make_async_copy(h_hbm.at[src_v.at[j0 + k]], buf.at[k],
                                      sg[k]).wait()
                pltpu.async_copy(buf.at[k], acc.at[dst_v.at[j0 + k]], ss[k],
                                 add=True)
            for k in range(NB):
                pltpu.make_async_copy(buf.at[k], acc.at[dst_v.at[j0 + k]],
                                      ss[k]).wait()
                pltpu.async_copy(h_hbm.at[src_v.at[j0 + NB + k]], buf.at[k],
                                 sg[k])

        j0 = (GRP - 1) * NB
        hs = []
        for k in range(NB):
            pltpu.make_async_copy(h_hbm.at[src_v.at[j0 + k]], buf.at[k],
                                  sg[k]).wait()
            hs.append(pltpu.async_copy(buf.at[k], acc.at[dst_v.at[j0 + k]],
                                       ss[k], add=True))
        for h in hs:
            h.wait()

    plsc.subcore_barrier()
    pltpu.sync_copy(
        acc.at[pl.ds(row0, RPS)], out_hbm.at[pl.ds(cid * NP + row0, RPS)]
    )


def _fc1_body(x_ref, w_ref, b_ref, o_ref):
    o_ref[...] = jnp.maximum(
        jnp.dot(x_ref[...], w_ref[...], precision=_HI,
                preferred_element_type=jnp.float32) + b_ref[...],
        0.0,
    )


_fc1 = pl.pallas_call(
    _fc1_body,
    grid=(NCH,),
    in_specs=[
        pl.BlockSpec((RC, D), lambda i: (i, 0)),
        pl.BlockSpec((D, D), lambda i: (0, 0)),
        pl.BlockSpec((1, D), lambda i: (0, 0)),
    ],
    out_specs=pl.BlockSpec((RC, D), lambda i: (i, 0)),
    out_shape=jax.ShapeDtypeStruct((N, D), jnp.float32),
)


def _post_body(final, p_ref, cp_ref, h_ref, lw_ref, lb_ref, rw_ref, g_ref,
               bb_ref, w2_ref, b2_ref, o_ref, z_ref):
    def step(j, carry):
        s, ss = carry
        r = pl.ds(j * RC, RC)
        r2 = pl.ds(NP + j * RC, RC)
        cnt = jnp.maximum(cp_ref[r, 0:1] + cp_ref[r2, 0:1], 1.0)
        agg = (p_ref[r, :] + p_ref[r2, :]) / cnt
        z = (jnp.dot(agg, lw_ref[...], precision=_HI,
                     preferred_element_type=jnp.float32)
             + lb_ref[...]
             + jnp.dot(h_ref[r, :], rw_ref[...], precision=_HI,
                       preferred_element_type=jnp.float32))
        z_ref[r, :] = z
        return (s + jnp.sum(z, axis=0, keepdims=True),
                ss + jnp.sum(z * z, axis=0, keepdims=True))

    init = (jnp.zeros((1, D), jnp.float32), jnp.zeros((1, D), jnp.float32))
    s, ss = lax.fori_loop(0, NCH, step, init)
    mu = s / N
    var = ss / N - mu * mu
    scale = g_ref[...] * lax.rsqrt(var + 1e-5)
    shift = bb_ref[...] - mu * scale

    @pl.loop(0, NCH)
    def _(j):
        r = pl.ds(j * RC, RC)
        hn = jnp.maximum(z_ref[r, :] * scale + shift, 0.0)
        if final:
            o = jnp.dot(hn, w2_ref[...], precision=_HI,
                        preferred_element_type=jnp.float32) + b2_ref[...]
            m = jnp.max(o, axis=1, keepdims=True)
            lse = m + jnp.log(jnp.sum(jnp.exp(o - m), axis=1, keepdims=True))
            o_ref[r, :] = o - lse
        else:
            o_ref[r, :] = hn


def _make_post(final):
    width = C if final else D
    return pl.pallas_call(
        functools.partial(_post_body, final),
        out_shape=jax.ShapeDtypeStruct((N, width), jnp.float32),
        scratch_shapes=[pltpu.VMEM((N, D), jnp.float32)],
    )


_post_mid = _make_post(False)
_post_final = _make_post(True)


def kernel(x, edge_index, fc1_W, fc1_b, lW0, lb0, rW0, g0, b0,
           lW1, lb1, rW1, g1, b1, lW2, lb2, rW2, g2, b2, fc2_W, fc2_b):
    src = edge_index[0]
    dst = edge_index[1]
    pad = EPAD - E
    srcp = jnp.concatenate(
        [src, jnp.zeros((pad,), jnp.int32)]).reshape(TOTALC, K)
    dstp = jnp.concatenate(
        [dst, jnp.full((pad,), N, jnp.int32)]).reshape(TOTALC, K)
    zeros_d = jnp.zeros((NP, D), jnp.float32)
    ones_d = jnp.ones((K, D), jnp.float32)

    cntp = _sc_count(dstp.reshape(NW, CH, K), ones_d, zeros_d)
    h = _fc1(x, fc1_W, fc1_b.reshape(1, D))

    layer_params = [(lW0, lb0, rW0, g0, b0),
                    (lW1, lb1, rW1, g1, b1),
                    (lW2, lb2, rW2, g2, b2)]
    out = None
    for l, (lW, lb, rW, g, b) in enumerate(layer_params):
        p = _sc_agg(h, srcp, dstp, zeros_d)
        post = _post_final if l == 2 else _post_mid
        res = post(p, cntp, h, lW, lb.reshape(1, D), rW, g.reshape(1, D),
                   b.reshape(1, D), fc2_W, fc2_b.reshape(1, C))
        if l == 2:
            out = res
        else:
            h = res
    return out


# speedup vs baseline: 1.6055x; 1.6055x over previous
import functools

import jax
import jax.numpy as jnp
from jax import lax
from jax.experimental import pallas as pl
from jax.experimental.pallas import tpu as pltpu
from jax.experimental.pallas import tpu_sc as plsc

N = 10000
E = 320000
D = 128
C = 16

NC = 2
NS = 16
NW = NC * NS
K = 128
CH = 80
QC = 8
EPAD = NW * CH * K
TOTALC = NW * CH

FAST_CID = 0
CHF = 152
CHS = 8
PF = CHF // QC
PS = CHS // QC
NP = 10112
RPS = NP // NS

RC = 1000
NCH = N // RC

_HI = jax.lax.Precision.HIGHEST

_mesh = plsc.VectorSubcoreMesh(
    core_axis_name="c", subcore_axis_name="s", num_cores=NC, num_subcores=NS
)


@functools.partial(
    pl.kernel,
    out_type=jax.ShapeDtypeStruct((NC * NP, D), jnp.float32),
    mesh=_mesh,
    scratch_types=[
        pltpu.VMEM((CH, K), jnp.int32),
        pltpu.VMEM((K, D), jnp.float32),
        pltpu.VMEM_SHARED((NP, D), jnp.float32),
        pltpu.SemaphoreType.DMA,
    ],
)
def _sc_count(dst_hbm, ones_hbm, z16_hbm, out_hbm, dst_v, ones_v, acc, sem):
    cid = lax.axis_index("c")
    sid = lax.axis_index("s")
    wid = sid * NC + cid
    pltpu.async_copy(dst_hbm.at[wid], dst_v, sem).wait()
    pltpu.async_copy(ones_hbm, ones_v, sem).wait()
    row0 = sid * RPS
    pltpu.async_copy(
        z16_hbm.at[pl.ds(row0, RPS)], acc.at[pl.ds(row0, RPS)], sem
    ).wait()
    plsc.subcore_barrier()

    @pl.loop(0, CH)
    def _(j):
        pltpu.sync_copy(ones_v, acc.at[dst_v.at[j]], add=True)

    plsc.subcore_barrier()
    pltpu.sync_copy(
        acc.at[pl.ds(row0, RPS)], out_hbm.at[pl.ds(cid * NP + row0, RPS)]
    )


NB = 2
GRP = QC // NB


@functools.partial(
    pl.kernel,
    out_type=jax.ShapeDtypeStruct((NC * NP, D), jnp.float32),
    mesh=_mesh,
    scratch_types=[
        pltpu.VMEM((QC, K), jnp.int32),
        pltpu.VMEM((QC, K), jnp.int32),
        pltpu.VMEM((NB, K, D), jnp.float32),
        pltpu.VMEM_SHARED((NP, D), jnp.float32),
        pltpu.SemaphoreType.DMA,
        pltpu.SemaphoreType.DMA,
        pltpu.SemaphoreType.DMA,
        pltpu.SemaphoreType.DMA,
        pltpu.SemaphoreType.DMA,
        pltpu.SemaphoreType.DMA,
        pltpu.SemaphoreType.DMA,
        pltpu.SemaphoreType.DMA,
        pltpu.SemaphoreType.DMA,
    ],
)
def _sc_agg(h_hbm, src_hbm, dst_hbm, zd_hbm, out_hbm,
            src_v, dst_v, buf, acc,
            sg0, sg1, sg2, sg3, ss0, ss1, ss2, ss3, sem):
    cid = lax.axis_index("c")
    sid = lax.axis_index("s")
    sg = (sg0, sg1, sg2, sg3)
    ss = (ss0, ss1, ss2, ss3)
    is_fast = cid == FAST_CID
    nph = jnp.where(is_fast, PF, PS)
    base = jnp.where(is_fast, sid * CHF, NS * CHF + sid * CHS)
    row0 = sid * RPS
    pltpu.async_copy(
        zd_hbm.at[pl.ds(row0, RPS)], acc.at[pl.ds(row0, RPS)], sem
    ).wait()
    plsc.subcore_barrier()

    @pl.loop(0, nph)
    def _(p):
        c0 = base + p * QC
        pltpu.async_copy(src_hbm.at[pl.ds(c0, QC)], src_v, sem)
        pltpu.async_copy(dst_hbm.at[pl.ds(c0, QC)], dst_v, sem)
        pltpu.make_async_copy(src_hbm.at[pl.ds(c0, QC)], src_v, sem).wait()
        pltpu.make_async_copy(dst_hbm.at[pl.ds(c0, QC)], dst_v, sem).wait()
        for k in range(NB):
            pltpu.async_copy(h_hbm.at[src_v.at[k]], buf.at[k], sg[k])

        @pl.loop(0, GRP - 1)
        def _(i):
            j0 = i * NB
            for k in range(NB):
                pltpu.make_async_copy(h_hbm.at[src_v.at[j0 + k]], buf.at[k],
                                      sg[k]).wait()
                pltpu.async_copy(buf.at[k], acc.at[dst_v.at[j0 + k]], ss[k],
                                 add=True)
            for k in range(NB):
                pltpu.make_async_copy(buf.at[k], acc.at[dst_v.at[j0 + k]],
                                      ss[k]).wait()
                pltpu.async_copy(h_hbm.at[src_v.at[j0 + NB + k]], buf.at[k],
                                 sg[k])

        j0 = (GRP - 1) * NB
        hs = []
        for k in range(NB):
            pltpu.make_async_copy(h_hbm.at[src_v.at[j0 + k]], buf.at[k],
                                  sg[k]).wait()
            hs.append(pltpu.async_copy(buf.at[k], acc.at[dst_v.at[j0 + k]],
                                       ss[k], add=True))
        for h in hs:
            h.wait()

    plsc.subcore_barrier()
    pltpu.sync_copy(
        acc.at[pl.ds(row0, RPS)], out_hbm.at[pl.ds(cid * NP + row0, RPS)]
    )


def _fc1_body(x_ref, w_ref, b_ref, o_ref):
    o_ref[...] = jnp.maximum(
        jnp.dot(x_ref[...], w_ref[...], precision=_HI,
                preferred_element_type=jnp.float32) + b_ref[...],
        0.0,
    )


_fc1 = pl.pallas_call(
    _fc1_body,
    grid=(NCH,),
    in_specs=[
        pl.BlockSpec((RC, D), lambda i: (i, 0)),
        pl.BlockSpec((D, D), lambda i: (0, 0)),
        pl.BlockSpec((1, D), lambda i: (0, 0)),
    ],
    out_specs=pl.BlockSpec((RC, D), lambda i: (i, 0)),
    out_shape=jax.ShapeDtypeStruct((N, D), jnp.float32),
)


def _post_body(final, p_ref, cp_ref, h_ref, lw_ref, lb_ref, rw_ref, g_ref,
               bb_ref, w2_ref, b2_ref, o_ref, z_ref):
    def step(j, carry):
        s, ss = carry
        r = pl.ds(j * RC, RC)
        r2 = pl.ds(NP + j * RC, RC)
        cnt = jnp.maximum(cp_ref[r, 0:1] + cp_ref[r2, 0:1], 1.0)
        agg = (p_ref[r, :] + p_ref[r2, :]) / cnt
        z = (jnp.dot(agg, lw_ref[...], precision=_HI,
                     preferred_element_type=jnp.float32)
             + lb_ref[...]
             + jnp.dot(h_ref[r, :], rw_ref[...], precision=_HI,
                       preferred_element_type=jnp.float32))
        z_ref[r, :] = z
        return (s + jnp.sum(z, axis=0, keepdims=True),
                ss + jnp.sum(z * z, axis=0, keepdims=True))

    init = (jnp.zeros((1, D), jnp.float32), jnp.zeros((1, D), jnp.float32))
    s, ss = lax.fori_loop(0, NCH, step, init)
    mu = s / N
    var = ss / N - mu * mu
    scale = g_ref[...] * lax.rsqrt(var + 1e-5)
    shift = bb_ref[...] - mu * scale

    @pl.loop(0, NCH)
    def _(j):
        r = pl.ds(j * RC, RC)
        hn = jnp.maximum(z_ref[r, :] * scale + shift, 0.0)
        if final:
            o = jnp.dot(hn, w2_ref[...], precision=_HI,
                        preferred_element_type=jnp.float32) + b2_ref[...]
            m = jnp.max(o, axis=1, keepdims=True)
            lse = m + jnp.log(jnp.sum(jnp.exp(o - m), axis=1, keepdims=True))
            o_ref[r, :] = o - lse
        else:
            o_ref[r, :] = hn


def _make_post(final):
    width = C if final else D
    return pl.pallas_call(
        functools.partial(_post_body, final),
        out_shape=jax.ShapeDtypeStruct((N, width), jnp.float32),
        scratch_shapes=[pltpu.VMEM((N, D), jnp.float32)],
    )


_post_mid = _make_post(False)
_post_final = _make_post(True)


def kernel(x, edge_index, fc1_W, fc1_b, lW0, lb0, rW0, g0, b0,
           lW1, lb1, rW1, g1, b1, lW2, lb2, rW2, g2, b2, fc2_W, fc2_b):
    src = edge_index[0]
    dst = edge_index[1]
    pad = EPAD - E
    srcp = jnp.concatenate(
        [src, jnp.zeros((pad,), jnp.int32)]).reshape(TOTALC, K)
    dstp = jnp.concatenate(
        [dst, jnp.full((pad,), N, jnp.int32)]).reshape(TOTALC, K)
    zeros_d = jnp.zeros((NP, D), jnp.float32)
    ones_d = jnp.ones((K, D), jnp.float32)

    cntp = _sc_count(dstp.reshape(NW, CH, K), ones_d, zeros_d)
    h = _fc1(x, fc1_W, fc1_b.reshape(1, D))

    layer_params = [(lW0, lb0, rW0, g0, b0),
                    (lW1, lb1, rW1, g1, b1),
                    (lW2, lb2, rW2, g2, b2)]
    out = None
    for l, (lW, lb, rW, g, b) in enumerate(layer_params):
        p = _sc_agg(h, srcp, dstp, zeros_d)
        post = _post_final if l == 2 else _post_mid
        res = post(p, cntp, h, lW, lb.reshape(1, D), rW, g.reshape(1, D),
                   b.reshape(1, D), fc2_W, fc2_b.reshape(1, C))
        if l == 2:
            out = res
        else:
            h = res
    return out
